# baseline (device time: 65767 ns/iter reference)
import os

import jax
import jax.numpy as jnp
from jax import lax
from jax.experimental import pallas as pl
from jax.experimental.pallas import tpu as pltpu

_NO_COMM = os.environ.get("KERNEL_NO_COMM") == "1"
_F32_MM = os.environ.get("KERNEL_F32_MM") == "1"

N_DEV = 16
LOG_N = 4
N_LAYERS = 3
N_STEPS = N_LAYERS * LOG_N


def _partner(my, k):
    if k == 1:
        return my - 2 * (my & 3) + 3
    return my ^ {0: 1, 2: 4, 3: 8}[k]


def kernel(x, Win0, Wout0, Win1, Wout1, Win2, Wout2):
    b, d = x.shape
    hdim = Win0.shape[1]

    def body(x_ref, win0, win1, win2, wout0, wout1, wout2,
             out_ref, winbuf, woutbuf, acc_ref, send_ref, comm_ref,
             win_dma_sems, wout_dma_sems, send_sems, recv_sems):
        my = lax.axis_index("i")
        win_hbm = [win0, win1, win2]
        wout_hbm = [wout0, wout1, wout2]

        win_cp = []
        wout_cp = []
        for l in range(N_LAYERS):
            cin = pltpu.make_async_copy(win_hbm[l], winbuf.at[l],
                                        win_dma_sems.at[l])
            cout = pltpu.make_async_copy(wout_hbm[l], woutbuf.at[l],
                                         wout_dma_sems.at[l])
            cin.start()
            cout.start()
            win_cp.append(cin)
            wout_cp.append(cout)

        cdt = jnp.float32 if _F32_MM else jnp.bfloat16
        x_val = x_ref[...].astype(cdt)
        for layer in range(N_LAYERS):
            win_cp[layer].wait()
            h = jnp.dot(x_val, winbuf[layer].astype(cdt),
                        preferred_element_type=jnp.float32)
            h = jnp.maximum(h, 0.0).astype(cdt)
            wout_cp[layer].wait()
            partial = jnp.dot(h, woutbuf[layer].astype(cdt),
                              preferred_element_type=jnp.float32)
            acc_ref[0] = partial
            send_ref[0] = partial.astype(jnp.bfloat16)
            for k in range(0 if _NO_COMM else LOG_N):
                step = layer * LOG_N + k
                cur, nxt = k % 2, (k + 1) % 2
                rdma = pltpu.make_async_remote_copy(
                    src_ref=send_ref.at[cur],
                    dst_ref=comm_ref.at[step],
                    send_sem=send_sems.at[step],
                    recv_sem=recv_sems.at[step],
                    device_id=(_partner(my, k),),
                    device_id_type=pl.DeviceIdType.MESH,
                )
                rdma.start()
                rdma.wait_recv()
                new_acc = acc_ref[cur] + comm_ref[step].astype(jnp.float32)
                acc_ref[nxt] = new_acc
                if k < LOG_N - 1:
                    send_ref[nxt] = new_acc.astype(jnp.bfloat16)
                rdma.wait_send()
            x_val = acc_ref[0 if _NO_COMM else LOG_N % 2].astype(cdt)
        out_ref[...] = acc_ref[0 if _NO_COMM else LOG_N % 2]

    return pl.pallas_call(
        body,
        out_shape=jax.ShapeDtypeStruct((b, d), jnp.float32),
        in_specs=[pl.BlockSpec(memory_space=pltpu.VMEM)]
        + [pl.BlockSpec(memory_space=pltpu.MemorySpace.HBM)] * 6,
        out_specs=pl.BlockSpec(memory_space=pltpu.VMEM),
        scratch_shapes=[
            pltpu.VMEM((N_LAYERS, d, hdim), jnp.float32),
            pltpu.VMEM((N_LAYERS, hdim, d), jnp.float32),
            pltpu.VMEM((2, b, d), jnp.float32),
            pltpu.VMEM((2, b, d), jnp.bfloat16),
            pltpu.VMEM((N_STEPS, b, d), jnp.bfloat16),
            pltpu.SemaphoreType.DMA((N_LAYERS,)),
            pltpu.SemaphoreType.DMA((N_LAYERS,)),
            pltpu.SemaphoreType.DMA((N_STEPS,)),
            pltpu.SemaphoreType.DMA((N_STEPS,)),
        ],
        compiler_params=pltpu.CompilerParams(
            vmem_limit_bytes=100 * 1024 * 1024,
        ),
    )(x, Win0, Win1, Win2, Wout0, Wout1, Wout2)


# device time: 52585 ns/iter; 1.2507x vs baseline; 1.2507x over previous
import os

import jax
import jax.numpy as jnp
from jax import lax
from jax.experimental import pallas as pl
from jax.experimental.pallas import tpu as pltpu

try:
    jax.config.update("jax_compilation_cache_dir", "/tmp/scband_jax_cache")
    jax.config.update("jax_persistent_cache_min_compile_time_secs", 1.0)
except Exception:
    pass

_NO_COMM = os.environ.get("KERNEL_NO_COMM") == "1"
_BF16_MM = os.environ.get("KERNEL_BF16_MM") == "1"

N_DEV = 16
LOG_N = 4
N_LAYERS = 3
N_STEPS = N_LAYERS * LOG_N
N_HALF = 2


def _partner(my, k):
    if k == 1:
        return my - 2 * (my & 3) + 3
    return my ^ {0: 1, 2: 4, 3: 8}[k]


def kernel(x, Win0, Wout0, Win1, Wout1, Win2, Wout2):
    b, d = x.shape
    hdim = Win0.shape[1]
    cols = d // N_HALF

    def body(x_ref, win0, win1, win2, wout0, wout1, wout2,
             out_ref, winbuf, woutbuf, xbuf, acc_ref, send_ref, comm_ref,
             win_dma_sems, wout_dma_sems, send_sems, recv_sems):
        my = lax.axis_index("i")
        win_hbm = [win0, win1, win2]
        wout_hbm = [wout0, wout1, wout2]

        win_cp = []
        wout_cp = []
        for l in range(N_LAYERS):
            cin = pltpu.make_async_copy(win_hbm[l], winbuf.at[l],
                                        win_dma_sems.at[l])
            cout = pltpu.make_async_copy(wout_hbm[l], woutbuf.at[l],
                                         wout_dma_sems.at[l])
            cin.start()
            cout.start()
            win_cp.append(cin)
            wout_cp.append(cout)

        if not _NO_COMM:
            barrier = pltpu.get_barrier_semaphore()
            for k in range(LOG_N):
                pl.semaphore_signal(
                    barrier, inc=1,
                    device_id=(_partner(my, k),),
                    device_id_type=pl.DeviceIdType.MESH,
                )

        def mk_rdma(step, half, slot):
            return pltpu.make_async_remote_copy(
                src_ref=send_ref.at[slot, half],
                dst_ref=comm_ref.at[step, half],
                send_sem=send_sems.at[step, half],
                recv_sem=recv_sems.at[step, half],
                device_id=(_partner(my, step % LOG_N),),
                device_id_type=pl.DeviceIdType.MESH,
            )

        cdt = jnp.bfloat16 if _BF16_MM else jnp.float32
        x_val = x_ref[...].astype(cdt)
        for layer in range(N_LAYERS):
            last_layer = layer == N_LAYERS - 1
            win_cp[layer].wait()
            h = jnp.dot(x_val, winbuf[layer].astype(cdt),
                        preferred_element_type=jnp.float32)
            h = jnp.maximum(h, 0.0).astype(cdt)
            wout_cp[layer].wait()

            if _NO_COMM:
                for half in range(N_HALF):
                    w = woutbuf[layer][:, half * cols:(half + 1) * cols]
                    p = jnp.dot(h, w.astype(cdt),
                                preferred_element_type=jnp.float32)
                    if last_layer:
                        out_ref[:, half * cols:(half + 1) * cols] = p
                    else:
                        xbuf[:, half * cols:(half + 1) * cols] = p
                x_val = xbuf[...].astype(cdt)
                continue

            rdmas = [None, None]
            for half in range(N_HALF):
                w = woutbuf[layer][:, half * cols:(half + 1) * cols]
                p = jnp.dot(h, w.astype(cdt),
                            preferred_element_type=jnp.float32)
                acc_ref[0, half] = p
                send_ref[0, half] = p.astype(jnp.bfloat16)
                if layer == 0 and half == 0:
                    pl.semaphore_wait(barrier, LOG_N)
                rdmas[half] = mk_rdma(layer * LOG_N, half, 0)
                rdmas[half].start()

            for k in range(LOG_N):
                step = layer * LOG_N + k
                cur, nxt = k % 2, (k + 1) % 2
                for half in range(N_HALF):
                    r = rdmas[half]
                    r.wait_recv()
                    new = (acc_ref[cur, half]
                           + comm_ref[step, half].astype(jnp.float32))
                    if k < LOG_N - 1:
                        acc_ref[nxt, half] = new
                        send_ref[nxt, half] = new.astype(jnp.bfloat16)
                        rdmas[half] = mk_rdma(step + 1, half, nxt)
                        rdmas[half].start()
                        r.wait_send()
                    else:
                        r.wait_send()
                        if last_layer:
                            out_ref[:, half * cols:(half + 1) * cols] = new
                        else:
                            xbuf[:, half * cols:(half + 1) * cols] = new
            if not last_layer:
                x_val = xbuf[...].astype(cdt)

    return pl.pallas_call(
        body,
        out_shape=jax.ShapeDtypeStruct((b, d), jnp.float32),
        in_specs=[pl.BlockSpec(memory_space=pltpu.VMEM)]
        + [pl.BlockSpec(memory_space=pltpu.MemorySpace.HBM)] * 6,
        out_specs=pl.BlockSpec(memory_space=pltpu.VMEM),
        scratch_shapes=[
            pltpu.VMEM((N_LAYERS, d, hdim), jnp.float32),
            pltpu.VMEM((N_LAYERS, hdim, d), jnp.float32),
            pltpu.VMEM((b, d), jnp.float32),
            pltpu.VMEM((2, N_HALF, b, cols), jnp.float32),
            pltpu.VMEM((2, N_HALF, b, cols), jnp.bfloat16),
            pltpu.VMEM((N_STEPS, N_HALF, b, cols), jnp.bfloat16),
            pltpu.SemaphoreType.DMA((N_LAYERS,)),
            pltpu.SemaphoreType.DMA((N_LAYERS,)),
            pltpu.SemaphoreType.DMA((N_STEPS, N_HALF)),
            pltpu.SemaphoreType.DMA((N_STEPS, N_HALF)),
        ],
        compiler_params=pltpu.CompilerParams(
            vmem_limit_bytes=100 * 1024 * 1024,
            collective_id=0,
        ),
    )(x, Win0, Win1, Win2, Wout0, Wout1, Wout2)
